# Initial kernel scaffold; baseline (speedup 1.0000x reference)
import jax
import jax.numpy as jnp
from jax.experimental import pallas as pl


def kernel(x):
    raise NotImplementedError("write your pallas kernel here")


# speedup vs baseline: 37.3628x; 37.3628x over previous
import jax
import jax.numpy as jnp
from jax.experimental import pallas as pl

_TL = 0.6
_L = 2.0


def _fcm_body(x_ref, out_ref):
    fm = x_ref[0]
    sq = jnp.sum(fm * fm, axis=1, keepdims=True)
    gram = jax.lax.dot_general(
        fm, fm,
        dimension_numbers=(((1,), (1,)), ((), ())),
        preferred_element_type=jnp.float32,
    )
    d2 = sq + sq.T - 2.0 * gram
    d = jnp.sqrt(jnp.maximum(d2, 1e-12))

    col = jax.lax.broadcasted_iota(jnp.int32, d.shape, 1)
    inf = jnp.float32(jnp.inf)

    mx = jnp.max(d, axis=1, keepdims=True)
    i1 = jnp.argmin(d, axis=1, keepdims=True)
    d_b = jnp.where(col == i1, inf, d)
    m2 = jnp.min(d_b, axis=1, keepdims=True)
    i2 = jnp.argmin(d_b, axis=1, keepdims=True)
    d_c = jnp.where(col == i2, inf, d_b)
    m3 = jnp.min(d_c, axis=1, keepdims=True)

    pred = jnp.where(
        m2 / m3 < _TL,
        2.0 / (1.0 + jnp.exp(m2)),
        2.0 / (1.0 + _L * jnp.exp(mx)),
    )
    out_ref[0] = pred


def kernel(x):
    b, h, w, c = x.shape
    hw = h * w
    fm = x.reshape(b, hw, c)
    pred = pl.pallas_call(
        _fcm_body,
        grid=(b,),
        in_specs=[pl.BlockSpec((1, hw, c), lambda i: (i, 0, 0))],
        out_specs=pl.BlockSpec((1, hw, 1), lambda i: (i, 0, 0)),
        out_shape=jax.ShapeDtypeStruct((b, hw, 1), jnp.float32),
    )(fm)
    return pred.reshape(b, h, w)
